# baseline (speedup 1.0000x reference)
import functools

import jax
import jax.numpy as jnp
from jax import lax
from jax.experimental import pallas as pl
from jax.experimental.pallas import tpu as pltpu
from jax.experimental.pallas import tpu_sc as plsc

BATCH = 16384
DIM = 128


_info = plsc.get_sparse_core_info()
_NC, _NS = _info.num_cores, _info.num_subcores
_NW = _NC * _NS
_BPW = BATCH // _NW


def _sc_gather_body(user_hbm, item_hbm, ut_hbm, it_hbm, u_out, i_out,
                    idx_v, rows_v, sem):
    wid = lax.axis_index("s") * _NC + lax.axis_index("c")
    base = wid * _BPW
    pltpu.sync_copy(user_hbm.at[pl.ds(base, _BPW)], idx_v)
    pltpu.async_copy(ut_hbm.at[idx_v], rows_v, sem).wait()
    pltpu.sync_copy(rows_v, u_out.at[pl.ds(base, _BPW)])
    pltpu.sync_copy(item_hbm.at[pl.ds(base, _BPW)], idx_v)
    pltpu.async_copy(it_hbm.at[idx_v], rows_v, sem).wait()
    pltpu.sync_copy(rows_v, i_out.at[pl.ds(base, _BPW)])


def _sc_gather(user, item, user_table, item_table):
    mesh = plsc.VectorSubcoreMesh(core_axis_name="c", subcore_axis_name="s")
    f = pl.kernel(
        _sc_gather_body,
        mesh=mesh,
        out_type=[
            jax.ShapeDtypeStruct((BATCH, DIM), jnp.float32),
            jax.ShapeDtypeStruct((BATCH, DIM), jnp.float32),
        ],
        scratch_types=[
            pltpu.VMEM((_BPW,), jnp.int32),
            pltpu.VMEM((_BPW, DIM), jnp.float32),
            pltpu.SemaphoreType.DMA,
        ],
    )
    return f(user, item, user_table, item_table)


_BM = 1024


def _mlp_body(u_ref, i_ref, w1u_ref, w1i_ref, b1_ref, w2_ref, b2_ref,
              w3_ref, b3_ref, wd_ref, bd_ref, out_ref):
    bf = jnp.bfloat16
    h = jnp.dot(u_ref[...].astype(bf), w1u_ref[...],
                preferred_element_type=jnp.float32)
    h += jnp.dot(i_ref[...].astype(bf), w1i_ref[...],
                 preferred_element_type=jnp.float32)
    h = jnp.maximum(h + b1_ref[...], 0.0).astype(bf)
    h = jnp.maximum(
        jnp.dot(h, w2_ref[...], preferred_element_type=jnp.float32)
        + b2_ref[...], 0.0).astype(bf)
    h = jnp.maximum(
        jnp.dot(h, w3_ref[...], preferred_element_type=jnp.float32)
        + b3_ref[...], 0.0)
    o = jnp.sum(h * wd_ref[...], axis=1, keepdims=True) + bd_ref[...]
    out_ref[...] = 1.0 / (1.0 + jnp.exp(-o))


def _mlp(u_emb, i_emb, W1, b1, W2, b2, W3, b3, Wd, bd):
    H1, H2, H3 = W1.shape[1], W2.shape[1], W3.shape[1]
    bf = jnp.bfloat16
    w1u = W1[:DIM].astype(bf)
    w1i = W1[DIM:].astype(bf)
    W2 = W2.astype(bf)
    W3 = W3.astype(bf)
    wdt = Wd.reshape(1, H3)
    grid = (BATCH // _BM,)
    zero = lambda i: (0, 0)
    out = pl.pallas_call(
        _mlp_body,
        grid=grid,
        in_specs=[
            pl.BlockSpec((_BM, DIM), lambda i: (i, 0)),
            pl.BlockSpec((_BM, DIM), lambda i: (i, 0)),
            pl.BlockSpec((DIM, H1), zero),
            pl.BlockSpec((DIM, H1), zero),
            pl.BlockSpec((1, H1), zero),
            pl.BlockSpec((H1, H2), zero),
            pl.BlockSpec((1, H2), zero),
            pl.BlockSpec((H2, H3), zero),
            pl.BlockSpec((1, H3), zero),
            pl.BlockSpec((1, H3), zero),
            pl.BlockSpec((1, 1), zero),
        ],
        out_specs=pl.BlockSpec((_BM, 1), lambda i: (i, 0)),
        out_shape=jax.ShapeDtypeStruct((BATCH, 1), jnp.float32),
    )(u_emb, i_emb, w1u, w1i, b1.reshape(1, H1), W2, b2.reshape(1, H2),
      W3, b3.reshape(1, H3), wdt, bd.reshape(1, 1))
    return out


def kernel(user, item, user_table, item_table, W1, b1, W2, b2, W3, b3, Wd, bd):
    u_emb, i_emb = _sc_gather(user, item, user_table, item_table)
    out = _mlp(u_emb, i_emb, W1, b1, W2, b2, W3, b3, Wd, bd)
    return out.reshape(-1)


# speedup vs baseline: 1.0451x; 1.0451x over previous
import functools

import jax
import jax.numpy as jnp
from jax import lax
from jax.experimental import pallas as pl
from jax.experimental.pallas import tpu as pltpu
from jax.experimental.pallas import tpu_sc as plsc

BATCH = 16384
DIM = 128


_info = plsc.get_sparse_core_info()
_NC, _NS = _info.num_cores, _info.num_subcores
_NW = _NC * _NS
_BPW = BATCH // _NW


def _sc_gather_body(user_hbm, item_hbm, ut_hbm, it_hbm, u_out, i_out,
                    idx_v, rows_v, sem):
    wid = lax.axis_index("s") * _NC + lax.axis_index("c")
    base = wid * _BPW
    pltpu.sync_copy(user_hbm.at[pl.ds(base, _BPW)], idx_v)
    pltpu.async_copy(ut_hbm.at[idx_v], rows_v, sem).wait()
    pltpu.sync_copy(rows_v, u_out.at[pl.ds(base, _BPW)])
    pltpu.sync_copy(item_hbm.at[pl.ds(base, _BPW)], idx_v)
    pltpu.async_copy(it_hbm.at[idx_v], rows_v, sem).wait()
    pltpu.sync_copy(rows_v, i_out.at[pl.ds(base, _BPW)])


def _sc_gather(user, item, user_table, item_table):
    mesh = plsc.VectorSubcoreMesh(core_axis_name="c", subcore_axis_name="s")
    f = pl.kernel(
        _sc_gather_body,
        mesh=mesh,
        out_type=[
            jax.ShapeDtypeStruct((BATCH, DIM), jnp.float32),
            jax.ShapeDtypeStruct((BATCH, DIM), jnp.float32),
        ],
        scratch_types=[
            pltpu.VMEM((_BPW,), jnp.int32),
            pltpu.VMEM((_BPW, DIM), jnp.float32),
            pltpu.SemaphoreType.DMA,
        ],
    )
    return f(user, item, user_table, item_table)


_BM = 2048


def _mlp_body(u_ref, i_ref, w1u_ref, w1i_ref, b1_ref, w2_ref, b2_ref,
              w3_ref, b3_ref, wd_ref, bd_ref, out_ref):
    h = jnp.dot(u_ref[...], w1u_ref[...], preferred_element_type=jnp.float32)
    h += jnp.dot(i_ref[...], w1i_ref[...], preferred_element_type=jnp.float32)
    h = jnp.maximum(h + b1_ref[...], 0.0)
    h = jnp.maximum(
        jnp.dot(h, w2_ref[...], preferred_element_type=jnp.float32)
        + b2_ref[...], 0.0)
    h = jnp.maximum(
        jnp.dot(h, w3_ref[...], preferred_element_type=jnp.float32)
        + b3_ref[...], 0.0)
    o = jnp.sum(h * wd_ref[...], axis=1, keepdims=True) + bd_ref[...]
    out_ref[...] = 1.0 / (1.0 + jnp.exp(-o))


def _mlp(u_emb, i_emb, W1, b1, W2, b2, W3, b3, Wd, bd):
    H1, H2, H3 = W1.shape[1], W2.shape[1], W3.shape[1]
    w1u = W1[:DIM]
    w1i = W1[DIM:]
    wdt = Wd.reshape(1, H3)
    grid = (BATCH // _BM,)
    zero = lambda i: (0, 0)
    out = pl.pallas_call(
        _mlp_body,
        grid=grid,
        in_specs=[
            pl.BlockSpec((_BM, DIM), lambda i: (i, 0)),
            pl.BlockSpec((_BM, DIM), lambda i: (i, 0)),
            pl.BlockSpec((DIM, H1), zero),
            pl.BlockSpec((DIM, H1), zero),
            pl.BlockSpec((1, H1), zero),
            pl.BlockSpec((H1, H2), zero),
            pl.BlockSpec((1, H2), zero),
            pl.BlockSpec((H2, H3), zero),
            pl.BlockSpec((1, H3), zero),
            pl.BlockSpec((1, H3), zero),
            pl.BlockSpec((1, 1), zero),
        ],
        out_specs=pl.BlockSpec((_BM, 1), lambda i: (i, 0)),
        out_shape=jax.ShapeDtypeStruct((BATCH, 1), jnp.float32),
    )(u_emb, i_emb, w1u, w1i, b1.reshape(1, H1), W2, b2.reshape(1, H2),
      W3, b3.reshape(1, H3), wdt, bd.reshape(1, 1))
    return out


def kernel(user, item, user_table, item_table, W1, b1, W2, b2, W3, b3, Wd, bd):
    u_emb, i_emb = _sc_gather(user, item, user_table, item_table)
    out = _mlp(u_emb, i_emb, W1, b1, W2, b2, W3, b3, Wd, bd)
    return out.reshape(-1)


# speedup vs baseline: 1.1490x; 1.0994x over previous
import jax
import jax.numpy as jnp
from jax import lax
from jax.experimental import pallas as pl
from jax.experimental.pallas import tpu as pltpu
from jax.experimental.pallas import tpu_sc as plsc

BATCH = 16384
DIM = 128


_info = plsc.get_sparse_core_info()
_NC, _NS = _info.num_cores, _info.num_subcores
_NW = _NC * _NS
_BPW = BATCH // _NW


def _sc_gather_body(user_hbm, item_hbm, ut_hbm, it_hbm, x_out,
                    idx_v, rows_v, sem):
    wid = lax.axis_index("s") * _NC + lax.axis_index("c")
    base = wid * _BPW
    pltpu.sync_copy(user_hbm.at[pl.ds(base, _BPW)], idx_v)
    pltpu.async_copy(ut_hbm.at[idx_v], rows_v, sem).wait()
    pltpu.sync_copy(rows_v, x_out.at[pl.ds(base, _BPW), pl.ds(0, DIM)])
    pltpu.sync_copy(item_hbm.at[pl.ds(base, _BPW)], idx_v)
    pltpu.async_copy(it_hbm.at[idx_v], rows_v, sem).wait()
    pltpu.sync_copy(rows_v, x_out.at[pl.ds(base, _BPW), pl.ds(DIM, DIM)])


def _sc_gather(user, item, user_table, item_table):
    mesh = plsc.VectorSubcoreMesh(core_axis_name="c", subcore_axis_name="s")
    f = pl.kernel(
        _sc_gather_body,
        mesh=mesh,
        out_type=jax.ShapeDtypeStruct((BATCH, 2 * DIM), jnp.float32),
        scratch_types=[
            pltpu.VMEM((_BPW,), jnp.int32),
            pltpu.VMEM((_BPW, DIM), jnp.float32),
            pltpu.SemaphoreType.DMA,
        ],
    )
    return f(user, item, user_table, item_table)


_BM = 2048


def _mlp_body(x_ref, w1_ref, b1_ref, w2_ref, b2_ref,
              w3_ref, b3_ref, wd_ref, bd_ref, out_ref):
    bf = jnp.bfloat16
    h = jnp.dot(x_ref[...].astype(bf), w1_ref[...],
                preferred_element_type=jnp.float32)
    h = jnp.maximum(h + b1_ref[...], 0.0).astype(bf)
    h = jnp.maximum(
        jnp.dot(h, w2_ref[...], preferred_element_type=jnp.float32)
        + b2_ref[...], 0.0).astype(bf)
    h = jnp.maximum(
        jnp.dot(h, w3_ref[...], preferred_element_type=jnp.float32)
        + b3_ref[...], 0.0)
    o = jnp.sum(h * wd_ref[...], axis=1, keepdims=True) + bd_ref[...]
    out_ref[...] = 1.0 / (1.0 + jnp.exp(-o))


def _mlp(x, W1, b1, W2, b2, W3, b3, Wd, bd):
    H1, H2, H3 = W1.shape[1], W2.shape[1], W3.shape[1]
    bf = jnp.bfloat16
    wdt = Wd.reshape(1, H3)
    grid = (BATCH // _BM,)
    zero = lambda i: (0, 0)
    out = pl.pallas_call(
        _mlp_body,
        grid=grid,
        in_specs=[
            pl.BlockSpec((_BM, 2 * DIM), lambda i: (i, 0)),
            pl.BlockSpec((2 * DIM, H1), zero),
            pl.BlockSpec((1, H1), zero),
            pl.BlockSpec((H1, H2), zero),
            pl.BlockSpec((1, H2), zero),
            pl.BlockSpec((H2, H3), zero),
            pl.BlockSpec((1, H3), zero),
            pl.BlockSpec((1, H3), zero),
            pl.BlockSpec((1, 1), zero),
        ],
        out_specs=pl.BlockSpec((_BM, 1), lambda i: (i, 0)),
        out_shape=jax.ShapeDtypeStruct((BATCH, 1), jnp.float32),
    )(x, W1.astype(bf), b1.reshape(1, H1), W2.astype(bf), b2.reshape(1, H2),
      W3.astype(bf), b3.reshape(1, H3), wdt, bd.reshape(1, 1))
    return out


def kernel(user, item, user_table, item_table, W1, b1, W2, b2, W3, b3, Wd, bd):
    x = _sc_gather(user, item, user_table, item_table)
    out = _mlp(x, W1, b1, W2, b2, W3, b3, Wd, bd)
    return out.reshape(-1)


# speedup vs baseline: 1.1523x; 1.0028x over previous
import jax
import jax.numpy as jnp
from jax import lax
from jax.experimental import pallas as pl
from jax.experimental.pallas import tpu as pltpu
from jax.experimental.pallas import tpu_sc as plsc

BATCH = 16384
DIM = 128


_info = plsc.get_sparse_core_info()
_NC, _NS = _info.num_cores, _info.num_subcores
_NW = _NC * _NS


def _make_sc_gather_body(n_rows):
    bpw = n_rows // _NW

    def body(user_hbm, item_hbm, ut_hbm, it_hbm, x_out, idx_v, rows_v, sem):
        wid = lax.axis_index("s") * _NC + lax.axis_index("c")
        base = wid * bpw
        pltpu.sync_copy(user_hbm.at[pl.ds(base, bpw)], idx_v)
        pltpu.async_copy(ut_hbm.at[idx_v], rows_v, sem).wait()
        pltpu.sync_copy(rows_v, x_out.at[pl.ds(base, bpw), pl.ds(0, DIM)])
        pltpu.sync_copy(item_hbm.at[pl.ds(base, bpw)], idx_v)
        pltpu.async_copy(it_hbm.at[idx_v], rows_v, sem).wait()
        pltpu.sync_copy(rows_v, x_out.at[pl.ds(base, bpw), pl.ds(DIM, DIM)])

    return body


def _sc_gather(user, item, user_table, item_table):
    n_rows = user.shape[0]
    bpw = n_rows // _NW
    mesh = plsc.VectorSubcoreMesh(core_axis_name="c", subcore_axis_name="s")
    f = pl.kernel(
        _make_sc_gather_body(n_rows),
        mesh=mesh,
        out_type=jax.ShapeDtypeStruct((n_rows, 2 * DIM), jnp.float32),
        scratch_types=[
            pltpu.VMEM((bpw,), jnp.int32),
            pltpu.VMEM((bpw, DIM), jnp.float32),
            pltpu.SemaphoreType.DMA,
        ],
    )
    return f(user, item, user_table, item_table)


_BM = 2048


def _mlp_body(x_ref, w1_ref, b1_ref, w2_ref, b2_ref,
              w3_ref, b3_ref, wd_ref, bd_ref, out_ref):
    bf = jnp.bfloat16
    h = jnp.dot(x_ref[...].astype(bf), w1_ref[...],
                preferred_element_type=jnp.float32)
    h = jnp.maximum(h + b1_ref[...], 0.0).astype(bf)
    h = jnp.maximum(
        jnp.dot(h, w2_ref[...], preferred_element_type=jnp.float32)
        + b2_ref[...], 0.0).astype(bf)
    h = jnp.maximum(
        jnp.dot(h, w3_ref[...], preferred_element_type=jnp.float32)
        + b3_ref[...], 0.0)
    o = jnp.sum(h * wd_ref[...], axis=1, keepdims=True) + bd_ref[...]
    out_ref[...] = 1.0 / (1.0 + jnp.exp(-o))


def _mlp(x, W1, b1, W2, b2, W3, b3, Wd, bd):
    n_rows = x.shape[0]
    H1, H2, H3 = W1.shape[1], W2.shape[1], W3.shape[1]
    bf = jnp.bfloat16
    wdt = Wd.reshape(1, H3)
    grid = (n_rows // _BM,)
    zero = lambda i: (0, 0)
    out = pl.pallas_call(
        _mlp_body,
        grid=grid,
        in_specs=[
            pl.BlockSpec((_BM, 2 * DIM), lambda i: (i, 0)),
            pl.BlockSpec((2 * DIM, H1), zero),
            pl.BlockSpec((1, H1), zero),
            pl.BlockSpec((H1, H2), zero),
            pl.BlockSpec((1, H2), zero),
            pl.BlockSpec((H2, H3), zero),
            pl.BlockSpec((1, H3), zero),
            pl.BlockSpec((1, H3), zero),
            pl.BlockSpec((1, 1), zero),
        ],
        out_specs=pl.BlockSpec((_BM, 1), lambda i: (i, 0)),
        out_shape=jax.ShapeDtypeStruct((n_rows, 1), jnp.float32),
    )(x, W1.astype(bf), b1.reshape(1, H1), W2.astype(bf), b2.reshape(1, H2),
      W3.astype(bf), b3.reshape(1, H3), wdt, bd.reshape(1, 1))
    return out


def kernel(user, item, user_table, item_table, W1, b1, W2, b2, W3, b3, Wd, bd):
    half = BATCH // 2
    xa = _sc_gather(user[:half], item[:half], user_table, item_table)
    xb = _sc_gather(user[half:], item[half:], user_table, item_table)
    oa = _mlp(xa, W1, b1, W2, b2, W3, b3, Wd, bd)
    ob = _mlp(xb, W1, b1, W2, b2, W3, b3, Wd, bd)
    return jnp.concatenate([oa, ob], axis=0).reshape(-1)
